# baseline (speedup 1.0000x reference)
import functools

import jax
import jax.numpy as jnp
from jax import lax
from jax.experimental import pallas as pl
from jax.experimental.pallas import tpu as pltpu
from jax.experimental.pallas import tpu_sc as plsc

N_REGION = 10000
N_SLOT = 8
HIDDEN = 256
LANES = 16
CHUNKS = HIDDEN // LANES

N_ROWS = N_REGION * N_SLOT
TC_BLOCK_ROWS = 2000
TC_GRID = N_ROWS // TC_BLOCK_ROWS


def _splat_total(v):
    c = plsc.cumsum(v)
    idx = jnp.full((LANES, 1), LANES - 1, jnp.int32)
    return lax.gather(
        c, idx,
        lax.GatherDimensionNumbers(offset_dims=(), collapsed_slice_dims=(0,),
                                   start_index_map=(0,)),
        (1,), mode=lax.GatherScatterMode.PROMISE_IN_BOUNDS)


def _sc_body(mem_hbm, idx_hbm, o_emb_hbm, w1_hbm, w2_hbm, upd_hbm,
             idx_v, rows_v, o_emb_v, w1_v, w2_v, upd_v, sem):
    wid = lax.axis_index("s") * 2 + lax.axis_index("c")

    @pl.when(wid == 0)
    def _():
        pltpu.sync_copy(idx_hbm, idx_v)
        pltpu.async_copy(mem_hbm.at[idx_v], rows_v, sem).wait()
        pltpu.sync_copy(o_emb_hbm, o_emb_v)
        pltpu.sync_copy(w1_hbm, w1_v)
        pltpu.sync_copy(w2_hbm, w2_v)

        acc = jnp.zeros((LANES,), jnp.float32)
        for c in range(CHUNKS):
            sl = pl.ds(c * LANES, LANES)
            acc = acc + o_emb_v[sl] * w1_v[sl]
        c0 = _splat_total(acc)

        for s in range(N_SLOT):
            acc = jnp.zeros((LANES,), jnp.float32)
            for c in range(CHUNKS):
                sl = pl.ds(c * LANES, LANES)
                acc = acc + rows_v[s, sl] * w2_v[sl]
            d = _splat_total(acc) + c0
            g = 1.0 / (1.0 + jnp.exp(-d))
            for c in range(CHUNKS):
                sl = pl.ds(c * LANES, LANES)
                upd_v[s, sl] = rows_v[s, sl] * (1.0 - g) + o_emb_v[sl] * g

        pltpu.sync_copy(upd_v, upd_hbm)


@functools.lru_cache(maxsize=1)
def _get_sc_update():
    return pl.kernel(
        _sc_body,
        out_type=jax.ShapeDtypeStruct((N_SLOT, HIDDEN), jnp.float32),
        mesh=plsc.VectorSubcoreMesh(core_axis_name="c", subcore_axis_name="s"),
        compiler_params=pltpu.CompilerParams(needs_layout_passes=False),
        scratch_types=[
            pltpu.VMEM((N_SLOT,), jnp.int32),
            pltpu.VMEM((N_SLOT, HIDDEN), jnp.float32),
            pltpu.VMEM((HIDDEN,), jnp.float32),
            pltpu.VMEM((HIDDEN,), jnp.float32),
            pltpu.VMEM((HIDDEN,), jnp.float32),
            pltpu.VMEM((N_SLOT, HIDDEN), jnp.float32),
            pltpu.SemaphoreType.DMA,
        ],
    )


N_COPY_CHUNKS = 16
COPY_CHUNK_ROWS = N_ROWS // N_COPY_CHUNKS


def _tc_copy_body(mem_ref, out_ref, sem):
    copies = [
        pltpu.make_async_copy(
            mem_ref.at[pl.ds(i * COPY_CHUNK_ROWS, COPY_CHUNK_ROWS)],
            out_ref.at[pl.ds(i * COPY_CHUNK_ROWS, COPY_CHUNK_ROWS)],
            sem)
        for i in range(N_COPY_CHUNKS)
    ]
    for c in copies:
        c.start()
    for c in copies:
        c.wait()


_tc_copy = pl.pallas_call(
    _tc_copy_body,
    in_specs=[pl.BlockSpec(memory_space=pltpu.HBM)],
    out_specs=pl.BlockSpec(memory_space=pltpu.HBM),
    scratch_shapes=[pltpu.SemaphoreType.DMA],
    out_shape=jax.ShapeDtypeStruct((N_ROWS, HIDDEN), jnp.float32),
)


def _tc_scatter_body(o_rg_ref, upd_ref, copied_ref, out_ref):
    del copied_ref
    out_ref[...] = upd_ref[...]


_tc_scatter = pl.pallas_call(
    _tc_scatter_body,
    grid_spec=pltpu.PrefetchScalarGridSpec(
        num_scalar_prefetch=1,
        grid=(1,),
        in_specs=[
            pl.BlockSpec((N_SLOT, HIDDEN), lambda i, s: (0, 0)),
            pl.BlockSpec(memory_space=pltpu.HBM),
        ],
        out_specs=pl.BlockSpec((N_SLOT, HIDDEN), lambda i, s: (s[0], 0)),
    ),
    out_shape=jax.ShapeDtypeStruct((N_ROWS, HIDDEN), jnp.float32),
    input_output_aliases={2: 0},
)


def kernel(memory, o_emb, forget_W, o_rg):
    o_rg = jnp.asarray(o_rg, jnp.int32)
    mem_rows = memory.reshape(N_REGION * N_SLOT, HIDDEN)
    idx = o_rg * N_SLOT + jnp.arange(N_SLOT, dtype=jnp.int32)
    w1 = forget_W[:HIDDEN, 0]
    w2 = forget_W[HIDDEN:, 0]
    upd = _get_sc_update()(mem_rows, idx, o_emb, w1, w2)
    copied = _tc_copy(mem_rows)
    out2d = _tc_scatter(o_rg.reshape(1), upd, copied)
    return out2d.reshape(N_REGION, N_SLOT, HIDDEN)


# speedup vs baseline: 32.9707x; 32.9707x over previous
import functools

import jax
import jax.numpy as jnp
from jax import lax
from jax.experimental import pallas as pl
from jax.experimental.pallas import tpu as pltpu
from jax.experimental.pallas import tpu_sc as plsc

N_REGION = 10000
N_SLOT = 8
HIDDEN = 256
LANES = 16
CHUNKS = HIDDEN // LANES

N_ROWS = N_REGION * N_SLOT
TC_BLOCK_ROWS = 2000
TC_GRID = N_ROWS // TC_BLOCK_ROWS


def _splat_total(v):
    c = plsc.cumsum(v)
    idx = jnp.full((LANES, 1), LANES - 1, jnp.int32)
    return lax.gather(
        c, idx,
        lax.GatherDimensionNumbers(offset_dims=(), collapsed_slice_dims=(0,),
                                   start_index_map=(0,)),
        (1,), mode=lax.GatherScatterMode.PROMISE_IN_BOUNDS)


def _sc_body(mem_hbm, idx_hbm, o_emb_hbm, w1_hbm, w2_hbm, upd_hbm,
             idx_v, rows_v, o_emb_v, w1_v, w2_v, upd_v, sem):
    wid = lax.axis_index("s") * 2 + lax.axis_index("c")

    @pl.when(wid == 0)
    def _():
        pltpu.sync_copy(idx_hbm, idx_v)
        pltpu.async_copy(mem_hbm.at[idx_v], rows_v, sem).wait()
        pltpu.sync_copy(o_emb_hbm, o_emb_v)
        pltpu.sync_copy(w1_hbm, w1_v)
        pltpu.sync_copy(w2_hbm, w2_v)

        acc = jnp.zeros((LANES,), jnp.float32)
        for c in range(CHUNKS):
            sl = pl.ds(c * LANES, LANES)
            acc = acc + o_emb_v[sl] * w1_v[sl]
        c0 = _splat_total(acc)

        for s in range(N_SLOT):
            acc = jnp.zeros((LANES,), jnp.float32)
            for c in range(CHUNKS):
                sl = pl.ds(c * LANES, LANES)
                acc = acc + rows_v[s, sl] * w2_v[sl]
            d = _splat_total(acc) + c0
            g = 1.0 / (1.0 + jnp.exp(-d))
            for c in range(CHUNKS):
                sl = pl.ds(c * LANES, LANES)
                upd_v[s, sl] = rows_v[s, sl] * (1.0 - g) + o_emb_v[sl] * g

        pltpu.sync_copy(upd_v, upd_hbm)


@functools.lru_cache(maxsize=1)
def _get_sc_update():
    return pl.kernel(
        _sc_body,
        out_type=jax.ShapeDtypeStruct((N_SLOT, HIDDEN), jnp.float32),
        mesh=plsc.VectorSubcoreMesh(core_axis_name="c", subcore_axis_name="s"),
        compiler_params=pltpu.CompilerParams(needs_layout_passes=False),
        scratch_types=[
            pltpu.VMEM((N_SLOT,), jnp.int32),
            pltpu.VMEM((N_SLOT, HIDDEN), jnp.float32),
            pltpu.VMEM((HIDDEN,), jnp.float32),
            pltpu.VMEM((HIDDEN,), jnp.float32),
            pltpu.VMEM((HIDDEN,), jnp.float32),
            pltpu.VMEM((N_SLOT, HIDDEN), jnp.float32),
            pltpu.SemaphoreType.DMA,
        ],
    )


def _tc_copy_body(mem_ref, out_ref):
    out_ref[...] = mem_ref[...]


_tc_copy = pl.pallas_call(
    _tc_copy_body,
    grid=(TC_GRID,),
    in_specs=[pl.BlockSpec((TC_BLOCK_ROWS, HIDDEN), lambda i: (i, 0))],
    out_specs=pl.BlockSpec((TC_BLOCK_ROWS, HIDDEN), lambda i: (i, 0)),
    out_shape=jax.ShapeDtypeStruct((N_ROWS, HIDDEN), jnp.float32),
)


def _tc_scatter_body(o_rg_ref, upd_ref, copied_ref, out_ref):
    del copied_ref
    out_ref[...] = upd_ref[...]


_tc_scatter = pl.pallas_call(
    _tc_scatter_body,
    grid_spec=pltpu.PrefetchScalarGridSpec(
        num_scalar_prefetch=1,
        grid=(1,),
        in_specs=[
            pl.BlockSpec((N_SLOT, HIDDEN), lambda i, s: (0, 0)),
            pl.BlockSpec(memory_space=pltpu.HBM),
        ],
        out_specs=pl.BlockSpec((N_SLOT, HIDDEN), lambda i, s: (s[0], 0)),
    ),
    out_shape=jax.ShapeDtypeStruct((N_ROWS, HIDDEN), jnp.float32),
    input_output_aliases={2: 0},
)


def kernel(memory, o_emb, forget_W, o_rg):
    o_rg = jnp.asarray(o_rg, jnp.int32)
    mem_rows = memory.reshape(N_REGION * N_SLOT, HIDDEN)
    idx = o_rg * N_SLOT + jnp.arange(N_SLOT, dtype=jnp.int32)
    w1 = forget_W[:HIDDEN, 0]
    w2 = forget_W[HIDDEN:, 0]
    upd = _get_sc_update()(mem_rows, idx, o_emb, w1, w2)
    copied = _tc_copy(mem_rows)
    out2d = _tc_scatter(o_rg.reshape(1), upd, copied)
    return out2d.reshape(N_REGION, N_SLOT, HIDDEN)


# speedup vs baseline: 35.2362x; 1.0687x over previous
import functools

import jax
import jax.numpy as jnp
from jax import lax
from jax.experimental import pallas as pl
from jax.experimental.pallas import tpu as pltpu
from jax.experimental.pallas import tpu_sc as plsc

N_REGION = 10000
N_SLOT = 8
HIDDEN = 256
LANES = 16
CHUNKS = HIDDEN // LANES

N_ROWS = N_REGION * N_SLOT
TC_BLOCK_ROWS = 4000
TC_GRID = N_ROWS // TC_BLOCK_ROWS


def _splat_total(v):
    c = plsc.cumsum(v)
    idx = jnp.full((LANES, 1), LANES - 1, jnp.int32)
    return lax.gather(
        c, idx,
        lax.GatherDimensionNumbers(offset_dims=(), collapsed_slice_dims=(0,),
                                   start_index_map=(0,)),
        (1,), mode=lax.GatherScatterMode.PROMISE_IN_BOUNDS)


def _sc_body(mem_hbm, idx_hbm, o_emb_hbm, w1_hbm, w2_hbm, upd_hbm,
             idx_v, rows_v, o_emb_v, w1_v, w2_v, upd_v, sem):
    wid = lax.axis_index("s") * 2 + lax.axis_index("c")

    @pl.when(wid == 0)
    def _():
        pltpu.sync_copy(idx_hbm, idx_v)
        pltpu.async_copy(mem_hbm.at[idx_v], rows_v, sem).wait()
        pltpu.sync_copy(o_emb_hbm, o_emb_v)
        pltpu.sync_copy(w1_hbm, w1_v)
        pltpu.sync_copy(w2_hbm, w2_v)

        acc = jnp.zeros((LANES,), jnp.float32)
        for c in range(CHUNKS):
            sl = pl.ds(c * LANES, LANES)
            acc = acc + o_emb_v[sl] * w1_v[sl]
        c0 = _splat_total(acc)

        for s in range(N_SLOT):
            acc = jnp.zeros((LANES,), jnp.float32)
            for c in range(CHUNKS):
                sl = pl.ds(c * LANES, LANES)
                acc = acc + rows_v[s, sl] * w2_v[sl]
            d = _splat_total(acc) + c0
            g = 1.0 / (1.0 + jnp.exp(-d))
            for c in range(CHUNKS):
                sl = pl.ds(c * LANES, LANES)
                upd_v[s, sl] = rows_v[s, sl] * (1.0 - g) + o_emb_v[sl] * g

        pltpu.sync_copy(upd_v, upd_hbm)


@functools.lru_cache(maxsize=1)
def _get_sc_update():
    return pl.kernel(
        _sc_body,
        out_type=jax.ShapeDtypeStruct((N_SLOT, HIDDEN), jnp.float32),
        mesh=plsc.VectorSubcoreMesh(core_axis_name="c", subcore_axis_name="s"),
        compiler_params=pltpu.CompilerParams(needs_layout_passes=False),
        scratch_types=[
            pltpu.VMEM((N_SLOT,), jnp.int32),
            pltpu.VMEM((N_SLOT, HIDDEN), jnp.float32),
            pltpu.VMEM((HIDDEN,), jnp.float32),
            pltpu.VMEM((HIDDEN,), jnp.float32),
            pltpu.VMEM((HIDDEN,), jnp.float32),
            pltpu.VMEM((N_SLOT, HIDDEN), jnp.float32),
            pltpu.SemaphoreType.DMA,
        ],
    )


def _tc_copy_body(mem_ref, out_ref):
    out_ref[...] = mem_ref[...]


_tc_copy = pl.pallas_call(
    _tc_copy_body,
    grid=(TC_GRID,),
    in_specs=[pl.BlockSpec((TC_BLOCK_ROWS, HIDDEN), lambda i: (i, 0))],
    out_specs=pl.BlockSpec((TC_BLOCK_ROWS, HIDDEN), lambda i: (i, 0)),
    out_shape=jax.ShapeDtypeStruct((N_ROWS, HIDDEN), jnp.float32),
)


def _tc_scatter_body(o_rg_ref, upd_ref, copied_ref, out_ref):
    del copied_ref
    out_ref[...] = upd_ref[...]


_tc_scatter = pl.pallas_call(
    _tc_scatter_body,
    grid_spec=pltpu.PrefetchScalarGridSpec(
        num_scalar_prefetch=1,
        grid=(1,),
        in_specs=[
            pl.BlockSpec((N_SLOT, HIDDEN), lambda i, s: (0, 0)),
            pl.BlockSpec(memory_space=pltpu.HBM),
        ],
        out_specs=pl.BlockSpec((N_SLOT, HIDDEN), lambda i, s: (s[0], 0)),
    ),
    out_shape=jax.ShapeDtypeStruct((N_ROWS, HIDDEN), jnp.float32),
    input_output_aliases={2: 0},
)


def kernel(memory, o_emb, forget_W, o_rg):
    o_rg = jnp.asarray(o_rg, jnp.int32)
    mem_rows = memory.reshape(N_REGION * N_SLOT, HIDDEN)
    idx = o_rg * N_SLOT + jnp.arange(N_SLOT, dtype=jnp.int32)
    w1 = forget_W[:HIDDEN, 0]
    w2 = forget_W[HIDDEN:, 0]
    upd = _get_sc_update()(mem_rows, idx, o_emb, w1, w2)
    copied = _tc_copy(mem_rows)
    out2d = _tc_scatter(o_rg.reshape(1), upd, copied)
    return out2d.reshape(N_REGION, N_SLOT, HIDDEN)


# speedup vs baseline: 36.0550x; 1.0232x over previous
import functools

import jax
import jax.numpy as jnp
from jax import lax
from jax.experimental import pallas as pl
from jax.experimental.pallas import tpu as pltpu
from jax.experimental.pallas import tpu_sc as plsc

N_REGION = 10000
N_SLOT = 8
HIDDEN = 256
LANES = 16
CHUNKS = HIDDEN // LANES

N_ROWS = N_REGION * N_SLOT
TC_BLOCK_ROWS = 8000
TC_GRID = N_ROWS // TC_BLOCK_ROWS


def _splat_total(v):
    c = plsc.cumsum(v)
    idx = jnp.full((LANES, 1), LANES - 1, jnp.int32)
    return lax.gather(
        c, idx,
        lax.GatherDimensionNumbers(offset_dims=(), collapsed_slice_dims=(0,),
                                   start_index_map=(0,)),
        (1,), mode=lax.GatherScatterMode.PROMISE_IN_BOUNDS)


def _sc_body(mem_hbm, idx_hbm, o_emb_hbm, w1_hbm, w2_hbm, upd_hbm,
             idx_v, rows_v, o_emb_v, w1_v, w2_v, upd_v, sem):
    wid = lax.axis_index("s") * 2 + lax.axis_index("c")

    @pl.when(wid == 0)
    def _():
        pltpu.sync_copy(idx_hbm, idx_v)
        pltpu.async_copy(mem_hbm.at[idx_v], rows_v, sem).wait()
        pltpu.sync_copy(o_emb_hbm, o_emb_v)
        pltpu.sync_copy(w1_hbm, w1_v)
        pltpu.sync_copy(w2_hbm, w2_v)

        acc = jnp.zeros((LANES,), jnp.float32)
        for c in range(CHUNKS):
            sl = pl.ds(c * LANES, LANES)
            acc = acc + o_emb_v[sl] * w1_v[sl]
        c0 = _splat_total(acc)

        for s in range(N_SLOT):
            acc = jnp.zeros((LANES,), jnp.float32)
            for c in range(CHUNKS):
                sl = pl.ds(c * LANES, LANES)
                acc = acc + rows_v[s, sl] * w2_v[sl]
            d = _splat_total(acc) + c0
            g = 1.0 / (1.0 + jnp.exp(-d))
            for c in range(CHUNKS):
                sl = pl.ds(c * LANES, LANES)
                upd_v[s, sl] = rows_v[s, sl] * (1.0 - g) + o_emb_v[sl] * g

        pltpu.sync_copy(upd_v, upd_hbm)


@functools.lru_cache(maxsize=1)
def _get_sc_update():
    return pl.kernel(
        _sc_body,
        out_type=jax.ShapeDtypeStruct((N_SLOT, HIDDEN), jnp.float32),
        mesh=plsc.VectorSubcoreMesh(core_axis_name="c", subcore_axis_name="s"),
        compiler_params=pltpu.CompilerParams(needs_layout_passes=False),
        scratch_types=[
            pltpu.VMEM((N_SLOT,), jnp.int32),
            pltpu.VMEM((N_SLOT, HIDDEN), jnp.float32),
            pltpu.VMEM((HIDDEN,), jnp.float32),
            pltpu.VMEM((HIDDEN,), jnp.float32),
            pltpu.VMEM((HIDDEN,), jnp.float32),
            pltpu.VMEM((N_SLOT, HIDDEN), jnp.float32),
            pltpu.SemaphoreType.DMA,
        ],
    )


def _tc_copy_body(mem_ref, out_ref):
    out_ref[...] = mem_ref[...]


_tc_copy = pl.pallas_call(
    _tc_copy_body,
    grid=(TC_GRID,),
    in_specs=[pl.BlockSpec((TC_BLOCK_ROWS, HIDDEN), lambda i: (i, 0))],
    out_specs=pl.BlockSpec((TC_BLOCK_ROWS, HIDDEN), lambda i: (i, 0)),
    out_shape=jax.ShapeDtypeStruct((N_ROWS, HIDDEN), jnp.float32),
)


def _tc_scatter_body(o_rg_ref, upd_ref, copied_ref, out_ref):
    del copied_ref
    out_ref[...] = upd_ref[...]


_tc_scatter = pl.pallas_call(
    _tc_scatter_body,
    grid_spec=pltpu.PrefetchScalarGridSpec(
        num_scalar_prefetch=1,
        grid=(1,),
        in_specs=[
            pl.BlockSpec((N_SLOT, HIDDEN), lambda i, s: (0, 0)),
            pl.BlockSpec(memory_space=pltpu.HBM),
        ],
        out_specs=pl.BlockSpec((N_SLOT, HIDDEN), lambda i, s: (s[0], 0)),
    ),
    out_shape=jax.ShapeDtypeStruct((N_ROWS, HIDDEN), jnp.float32),
    input_output_aliases={2: 0},
)


def kernel(memory, o_emb, forget_W, o_rg):
    o_rg = jnp.asarray(o_rg, jnp.int32)
    mem_rows = memory.reshape(N_REGION * N_SLOT, HIDDEN)
    idx = o_rg * N_SLOT + jnp.arange(N_SLOT, dtype=jnp.int32)
    w1 = forget_W[:HIDDEN, 0]
    w2 = forget_W[HIDDEN:, 0]
    upd = _get_sc_update()(mem_rows, idx, o_emb, w1, w2)
    copied = _tc_copy(mem_rows)
    out2d = _tc_scatter(o_rg.reshape(1), upd, copied)
    return out2d.reshape(N_REGION, N_SLOT, HIDDEN)


# speedup vs baseline: 36.2827x; 1.0063x over previous
import functools

import jax
import jax.numpy as jnp
from jax import lax
from jax.experimental import pallas as pl
from jax.experimental.pallas import tpu as pltpu
from jax.experimental.pallas import tpu_sc as plsc

N_REGION = 10000
N_SLOT = 8
HIDDEN = 256
LANES = 16
CHUNKS = HIDDEN // LANES

N_ROWS = N_REGION * N_SLOT
TC_BLOCK_ROWS = 10000
TC_GRID = N_ROWS // TC_BLOCK_ROWS


def _splat_total(v):
    c = plsc.cumsum(v)
    idx = jnp.full((LANES, 1), LANES - 1, jnp.int32)
    return lax.gather(
        c, idx,
        lax.GatherDimensionNumbers(offset_dims=(), collapsed_slice_dims=(0,),
                                   start_index_map=(0,)),
        (1,), mode=lax.GatherScatterMode.PROMISE_IN_BOUNDS)


def _sc_body(mem_hbm, idx_hbm, o_emb_hbm, w1_hbm, w2_hbm, upd_hbm,
             idx_v, rows_v, o_emb_v, w1_v, w2_v, upd_v, sem):
    wid = lax.axis_index("s") * 2 + lax.axis_index("c")

    @pl.when(wid == 0)
    def _():
        pltpu.sync_copy(idx_hbm, idx_v)
        pltpu.async_copy(mem_hbm.at[idx_v], rows_v, sem).wait()
        pltpu.sync_copy(o_emb_hbm, o_emb_v)
        pltpu.sync_copy(w1_hbm, w1_v)
        pltpu.sync_copy(w2_hbm, w2_v)

        acc = jnp.zeros((LANES,), jnp.float32)
        for c in range(CHUNKS):
            sl = pl.ds(c * LANES, LANES)
            acc = acc + o_emb_v[sl] * w1_v[sl]
        c0 = _splat_total(acc)

        for s in range(N_SLOT):
            acc = jnp.zeros((LANES,), jnp.float32)
            for c in range(CHUNKS):
                sl = pl.ds(c * LANES, LANES)
                acc = acc + rows_v[s, sl] * w2_v[sl]
            d = _splat_total(acc) + c0
            g = 1.0 / (1.0 + jnp.exp(-d))
            for c in range(CHUNKS):
                sl = pl.ds(c * LANES, LANES)
                upd_v[s, sl] = rows_v[s, sl] * (1.0 - g) + o_emb_v[sl] * g

        pltpu.sync_copy(upd_v, upd_hbm)


@functools.lru_cache(maxsize=1)
def _get_sc_update():
    return pl.kernel(
        _sc_body,
        out_type=jax.ShapeDtypeStruct((N_SLOT, HIDDEN), jnp.float32),
        mesh=plsc.VectorSubcoreMesh(core_axis_name="c", subcore_axis_name="s"),
        compiler_params=pltpu.CompilerParams(needs_layout_passes=False),
        scratch_types=[
            pltpu.VMEM((N_SLOT,), jnp.int32),
            pltpu.VMEM((N_SLOT, HIDDEN), jnp.float32),
            pltpu.VMEM((HIDDEN,), jnp.float32),
            pltpu.VMEM((HIDDEN,), jnp.float32),
            pltpu.VMEM((HIDDEN,), jnp.float32),
            pltpu.VMEM((N_SLOT, HIDDEN), jnp.float32),
            pltpu.SemaphoreType.DMA,
        ],
    )


def _tc_copy_body(mem_ref, out_ref):
    out_ref[...] = mem_ref[...]


_tc_copy = pl.pallas_call(
    _tc_copy_body,
    grid=(TC_GRID,),
    in_specs=[pl.BlockSpec((TC_BLOCK_ROWS, HIDDEN), lambda i: (i, 0))],
    out_specs=pl.BlockSpec((TC_BLOCK_ROWS, HIDDEN), lambda i: (i, 0)),
    out_shape=jax.ShapeDtypeStruct((N_ROWS, HIDDEN), jnp.float32),
)


def _tc_scatter_body(o_rg_ref, upd_ref, copied_ref, out_ref):
    del copied_ref
    out_ref[...] = upd_ref[...]


_tc_scatter = pl.pallas_call(
    _tc_scatter_body,
    grid_spec=pltpu.PrefetchScalarGridSpec(
        num_scalar_prefetch=1,
        grid=(1,),
        in_specs=[
            pl.BlockSpec((N_SLOT, HIDDEN), lambda i, s: (0, 0)),
            pl.BlockSpec(memory_space=pltpu.HBM),
        ],
        out_specs=pl.BlockSpec((N_SLOT, HIDDEN), lambda i, s: (s[0], 0)),
    ),
    out_shape=jax.ShapeDtypeStruct((N_ROWS, HIDDEN), jnp.float32),
    input_output_aliases={2: 0},
)


def kernel(memory, o_emb, forget_W, o_rg):
    o_rg = jnp.asarray(o_rg, jnp.int32)
    mem_rows = memory.reshape(N_REGION * N_SLOT, HIDDEN)
    idx = o_rg * N_SLOT + jnp.arange(N_SLOT, dtype=jnp.int32)
    w1 = forget_W[:HIDDEN, 0]
    w2 = forget_W[HIDDEN:, 0]
    upd = _get_sc_update()(mem_rows, idx, o_emb, w1, w2)
    copied = _tc_copy(mem_rows)
    out2d = _tc_scatter(o_rg.reshape(1), upd, copied)
    return out2d.reshape(N_REGION, N_SLOT, HIDDEN)
